# baseline (speedup 1.0000x reference)
import functools

import jax
import jax.numpy as jnp
from jax import lax
from jax.experimental import pallas as pl
from jax.experimental.pallas import tpu as pltpu
from jax.experimental.pallas import tpu_sc as plsc

SIZE = 32
LANES = 16
NUM_CORES = 2
NUM_SUBCORES = 16
NUM_WORKERS = NUM_CORES * NUM_SUBCORES
IDX_CHUNK = 128


@functools.partial(jax.jit, static_argnums=0)
def _mf_sc(batch, uid, iid, uebd, iebd, ubias, ibias, mean16):
    b_per_w = batch // NUM_WORKERS
    n_chunks = b_per_w // IDX_CHUNK
    n_groups = b_per_w // LANES
    mesh = plsc.VectorSubcoreMesh(core_axis_name="c", subcore_axis_name="s")

    @functools.partial(
        pl.kernel,
        mesh=mesh,
        compiler_params=pltpu.CompilerParams(
            needs_layout_passes=False, use_tc_tiling_on_sc=False),
        out_type=jax.ShapeDtypeStruct((batch,), jnp.float32),
        scratch_types=[
            pltpu.VMEM((n_chunks, IDX_CHUNK), jnp.int32),
            pltpu.VMEM((n_chunks, IDX_CHUNK), jnp.int32),
            pltpu.VMEM((b_per_w, SIZE), jnp.float32),
            pltpu.VMEM((b_per_w, SIZE), jnp.float32),
            pltpu.VMEM((b_per_w,), jnp.float32),
            pltpu.VMEM((b_per_w,), jnp.float32),
            pltpu.VMEM((LANES,), jnp.float32),
            pltpu.VMEM((b_per_w,), jnp.float32),
            pltpu.SemaphoreType.DMA,
        ],
    )
    def mf(uid_hbm, iid_hbm, uebd_hbm, iebd_hbm, ubias_hbm, ibias_hbm,
           mean_hbm, out_hbm, uidx, iidx, urows, irows, ubv, ibv, meanv,
           outv, sem):
        wid = lax.axis_index("s") * NUM_CORES + lax.axis_index("c")
        pltpu.sync_copy(uid_hbm.at[wid], uidx)
        pltpu.sync_copy(iid_hbm.at[wid], iidx)
        pltpu.sync_copy(mean_hbm, meanv)
        copies = []
        for c in range(n_chunks):
            sl = pl.ds(c * IDX_CHUNK, IDX_CHUNK)
            copies.append(
                pltpu.async_copy(uebd_hbm.at[uidx.at[c]], urows.at[sl], sem))
            copies.append(
                pltpu.async_copy(iebd_hbm.at[iidx.at[c]], irows.at[sl], sem))
            copies.append(
                pltpu.async_copy(ubias_hbm.at[uidx.at[c]], ubv.at[sl], sem))
            copies.append(
                pltpu.async_copy(ibias_hbm.at[iidx.at[c]], ibv.at[sl], sem))
        for cp in copies:
            cp.wait()
        mean_vec = meanv[...]

        def group(g, carry):
            gbase = g * LANES
            rows_i = gbase + lax.iota(jnp.int32, LANES)
            acc = jnp.zeros((LANES,), jnp.float32)
            for j in range(SIZE):
                col = jnp.full((LANES,), j, jnp.int32)
                acc = acc + (plsc.load_gather(urows, [rows_i, col]) *
                             plsc.load_gather(irows, [rows_i, col]))
            res = (acc + ubv[pl.ds(gbase, LANES)] + ibv[pl.ds(gbase, LANES)]
                   + mean_vec)
            outv[pl.ds(gbase, LANES)] = res
            return carry

        lax.fori_loop(0, n_groups, group, 0)
        pltpu.sync_copy(outv, out_hbm.at[pl.ds(wid * b_per_w, b_per_w)])

    return mf(uid, iid, uebd, iebd, ubias, ibias, mean16)


def kernel(user_id, item_id, user_ebds, item_ebds, user_bias, item_bias, mean):
    batch = user_id.shape[0]
    b_per_w = batch // NUM_WORKERS
    uid = user_id.astype(jnp.int32).reshape(
        NUM_WORKERS, b_per_w // IDX_CHUNK, IDX_CHUNK)
    iid = item_id.astype(jnp.int32).reshape(
        NUM_WORKERS, b_per_w // IDX_CHUNK, IDX_CHUNK)
    mean16 = jnp.broadcast_to(mean.astype(jnp.float32), (LANES,))
    return _mf_sc(batch, uid, iid, user_ebds, item_ebds,
                  user_bias.reshape(-1), item_bias.reshape(-1), mean16)
